# Initial kernel scaffold; baseline (speedup 1.0000x reference)
import functools

import jax
import jax.numpy as jnp
from jax import lax
from jax.experimental import pallas as pl
from jax.experimental.pallas import tpu as pltpu
from jax.experimental.pallas import tpu_sc as plsc

D = 128
N = 100000
M = 25000

NC = 2
NS = 16
NW = NC * NS
CH = 128
RPW = 3200
NCH = RPW // CH
NP = NW * RPW

RB = 1000
EPS = 1e-5
_INV_SQRT2 = 0.7071067811865476


def _sc_gather(table, idx2d):
    mesh = plsc.VectorSubcoreMesh(
        core_axis_name="c", subcore_axis_name="s", num_cores=NC, num_subcores=NS
    )

    @functools.partial(
        pl.kernel,
        out_type=jax.ShapeDtypeStruct((NP, D), jnp.float32),
        mesh=mesh,
        scratch_types=[
            pltpu.VMEM((NCH, CH), jnp.int32),
            pltpu.VMEM((CH, D), jnp.float32),
            pltpu.SemaphoreType.DMA,
        ],
    )
    def k(table_hbm, idx_hbm, out_hbm, idx_v, rows_v, sem):
        wid = lax.axis_index("s") * NC + lax.axis_index("c")
        pltpu.sync_copy(idx_hbm.at[pl.ds(wid * NCH, NCH)], idx_v)

        def body(c, carry):
            pltpu.async_copy(table_hbm.at[idx_v.at[c]], rows_v, sem).wait()
            pltpu.sync_copy(rows_v, out_hbm.at[pl.ds(wid * RPW + c * CH, CH)])
            return carry

        lax.fori_loop(0, NCH, body, 0)

    return k(table, idx2d)


def _tc_mlp(xi, xs, w1a, w1b, b1, g1, be1, w2, b2, g2, be2):
    def body(xi_ref, xs_ref, w1a_ref, w1b_ref, b1_ref, g1_ref, be1_ref,
             w2_ref, b2_ref, g2_ref, be2_ref, out_ref):
        h = jnp.dot(xi_ref[...], w1a_ref[...], preferred_element_type=jnp.float32)
        h = h + jnp.dot(xs_ref[...], w1b_ref[...], preferred_element_type=jnp.float32)
        h = h + b1_ref[...]
        mu = jnp.mean(h, axis=-1, keepdims=True)
        c = h - mu
        var = jnp.mean(c * c, axis=-1, keepdims=True)
        h = c * lax.rsqrt(var + EPS) * g1_ref[...] + be1_ref[...]
        h = 0.5 * h * (1.0 + lax.erf(h * _INV_SQRT2))
        h = jnp.dot(h, w2_ref[...], preferred_element_type=jnp.float32) + b2_ref[...]
        mu = jnp.mean(h, axis=-1, keepdims=True)
        c = h - mu
        var = jnp.mean(c * c, axis=-1, keepdims=True)
        h = c * lax.rsqrt(var + EPS) * g2_ref[...] + be2_ref[...]
        out_ref[...] = 0.5 * h * (1.0 + lax.erf(h * _INV_SQRT2))

    row_spec = pl.BlockSpec((RB, D), lambda i: (i, 0))
    full = lambda shape: pl.BlockSpec(shape, lambda i: (0,) * len(shape))
    return pl.pallas_call(
        body,
        grid=(N // RB,),
        in_specs=[
            row_spec,
            row_spec,
            full((D, D)), full((D, D)), full((1, D)), full((1, D)), full((1, D)),
            full((D, D)), full((1, D)), full((1, D)), full((1, D)),
        ],
        out_specs=row_spec,
        out_shape=jax.ShapeDtypeStruct((N, D), jnp.float32),
    )(xi, xs, w1a, w1b, b1, g1, be1, w2, b2, g2, be2)


def kernel(x_down, x_skip, upsample_idx, W1, b1, g1, be1, W2, b2, g2, be2):
    idx = upsample_idx.astype(jnp.int32)
    idx = jnp.concatenate([idx, jnp.zeros((NP - N,), jnp.int32)])
    xi = _sc_gather(x_down, idx.reshape(NW * NCH, CH))
    return _tc_mlp(
        xi, x_skip,
        W1[:D], W1[D:],
        b1.reshape(1, D), g1.reshape(1, D), be1.reshape(1, D),
        W2, b2.reshape(1, D), g2.reshape(1, D), be2.reshape(1, D),
    )


# speedup vs baseline: 3.5001x; 3.5001x over previous
import functools

import jax
import jax.numpy as jnp
from jax import lax
from jax.experimental import pallas as pl
from jax.experimental.pallas import tpu as pltpu
from jax.experimental.pallas import tpu_sc as plsc

D = 128
N = 100000
M = 25000

NC = 2
NS = 16
NW = NC * NS
CH = 128
RPW = 3200
NCH = RPW // CH
NP = NW * RPW

RB = 1000
EPS = 1e-5
_INV_SQRT2 = 0.7071067811865476


def _sc_gather(table, idx2d):
    mesh = plsc.VectorSubcoreMesh(
        core_axis_name="c", subcore_axis_name="s", num_cores=NC, num_subcores=NS
    )

    @functools.partial(
        pl.kernel,
        out_type=jax.ShapeDtypeStruct((NP, D), jnp.float32),
        mesh=mesh,
        scratch_types=[
            pltpu.VMEM((NCH, CH), jnp.int32),
            pltpu.VMEM((CH, D), jnp.float32),
            pltpu.SemaphoreType.DMA,
        ],
    )
    def k(table_hbm, idx_hbm, out_hbm, idx_v, rows_v, sem):
        wid = lax.axis_index("s") * NC + lax.axis_index("c")
        pltpu.sync_copy(idx_hbm.at[wid], idx_v)

        def body(c, carry):
            pltpu.async_copy(table_hbm.at[idx_v.at[c]], rows_v, sem).wait()
            pltpu.sync_copy(rows_v, out_hbm.at[pl.ds(wid * RPW + c * CH, CH)])
            return carry

        lax.fori_loop(0, NCH, body, 0)

    return k(table, idx2d)


def _tc_mlp(xi, xs, w1a, w1b, b1, g1, be1, w2, b2, g2, be2):
    def body(xi_ref, xs_ref, w1a_ref, w1b_ref, b1_ref, g1_ref, be1_ref,
             w2_ref, b2_ref, g2_ref, be2_ref, out_ref):
        h = jnp.dot(xi_ref[...], w1a_ref[...], preferred_element_type=jnp.float32)
        h = h + jnp.dot(xs_ref[...], w1b_ref[...], preferred_element_type=jnp.float32)
        h = h + b1_ref[...]
        mu = jnp.mean(h, axis=-1, keepdims=True)
        c = h - mu
        var = jnp.mean(c * c, axis=-1, keepdims=True)
        h = c * lax.rsqrt(var + EPS) * g1_ref[...] + be1_ref[...]
        h = 0.5 * h * (1.0 + lax.erf(h * _INV_SQRT2))
        h = jnp.dot(h, w2_ref[...], preferred_element_type=jnp.float32) + b2_ref[...]
        mu = jnp.mean(h, axis=-1, keepdims=True)
        c = h - mu
        var = jnp.mean(c * c, axis=-1, keepdims=True)
        h = c * lax.rsqrt(var + EPS) * g2_ref[...] + be2_ref[...]
        out_ref[...] = 0.5 * h * (1.0 + lax.erf(h * _INV_SQRT2))

    row_spec = pl.BlockSpec((RB, D), lambda i: (i, 0))
    full = lambda shape: pl.BlockSpec(shape, lambda i: (0,) * len(shape))
    return pl.pallas_call(
        body,
        grid=(N // RB,),
        in_specs=[
            row_spec,
            row_spec,
            full((D, D)), full((D, D)), full((1, D)), full((1, D)), full((1, D)),
            full((D, D)), full((1, D)), full((1, D)), full((1, D)),
        ],
        out_specs=row_spec,
        out_shape=jax.ShapeDtypeStruct((N, D), jnp.float32),
    )(xi, xs, w1a, w1b, b1, g1, be1, w2, b2, g2, be2)


def kernel(x_down, x_skip, upsample_idx, W1, b1, g1, be1, W2, b2, g2, be2):
    idx = upsample_idx.astype(jnp.int32)
    idx = jnp.concatenate([idx, jnp.zeros((NP - N,), jnp.int32)])
    xi = _sc_gather(x_down, idx.reshape(NW, NCH, CH))
    return _tc_mlp(
        xi, x_skip,
        W1[:D], W1[D:],
        b1.reshape(1, D), g1.reshape(1, D), be1.reshape(1, D),
        W2, b2.reshape(1, D), g2.reshape(1, D), be2.reshape(1, D),
    )


# speedup vs baseline: 3.7428x; 1.0693x over previous
import functools

import jax
import jax.numpy as jnp
from jax import lax
from jax.experimental import pallas as pl
from jax.experimental.pallas import tpu as pltpu
from jax.experimental.pallas import tpu_sc as plsc

D = 128
N = 100000
M = 25000

NC = 2
NS = 16
NW = NC * NS
CH = 80
R = 5
G = 8
NCH = R * G
RPW = CH * NCH
NP = NW * RPW

RB = 1000
EPS = 1e-5
_INV_SQRT2 = 0.7071067811865476


def _sc_gather(table, idx2d):
    mesh = plsc.VectorSubcoreMesh(
        core_axis_name="c", subcore_axis_name="s", num_cores=NC, num_subcores=NS
    )

    @functools.partial(
        pl.kernel,
        out_type=jax.ShapeDtypeStruct((NP, D), jnp.float32),
        mesh=mesh,
        scratch_types=[
            pltpu.VMEM((NCH, CH), jnp.int32),
            [pltpu.VMEM((CH, D), jnp.float32)] * (2 * R),
            [pltpu.SemaphoreType.DMA] * 4,
        ],
    )
    def k(table_hbm, idx_hbm, out_hbm, idx_v, bufs, sems):
        buf_a, buf_b = bufs[:R], bufs[R:]
        sem_ga, sem_gb, sem_oa, sem_ob = sems
        wid = lax.axis_index("s") * NC + lax.axis_index("c")
        pltpu.sync_copy(idx_hbm.at[wid], idx_v)
        base = wid * RPW

        def fire_g(buf, c, sem):
            return pltpu.async_copy(table_hbm.at[idx_v.at[c]], buf, sem)

        def fire_o(buf, c, sem):
            return pltpu.async_copy(buf, out_hbm.at[pl.ds(base + c * CH, CH)], sem)

        def drain_g(buf, sem):
            pltpu.make_async_copy(table_hbm.at[pl.ds(0, CH)], buf, sem).wait()

        def drain_o(buf, sem):
            pltpu.make_async_copy(buf, out_hbm.at[pl.ds(base, CH)], sem).wait()

        for b in range(R):
            fire_g(buf_a[b], b, sem_ga)

        def body(u, carry):
            g0 = 2 * u
            g1 = g0 + 1
            for b in range(R):
                drain_g(buf_a[b], sem_ga)
            outs_a = [fire_o(buf_a[b], g0 * R + b, sem_oa) for b in range(R)]
            @pl.when(u > 0)
            def _():
                for b in range(R):
                    drain_o(buf_b[b], sem_ob)
            gb = [fire_g(buf_b[b], g1 * R + b, sem_gb) for b in range(R)]
            for d in gb:
                d.wait()
            for b in range(R):
                fire_o(buf_b[b], g1 * R + b, sem_ob)
            for d in outs_a:
                d.wait()
            @pl.when(u + 1 < G // 2)
            def _():
                for b in range(R):
                    fire_g(buf_a[b], (g0 + 2) * R + b, sem_ga)
            return carry

        lax.fori_loop(0, G // 2, body, 0)
        for b in range(R):
            drain_o(buf_b[b], sem_ob)

    return k(table, idx2d)


def _tc_mlp(xi, xs, w1a, w1b, b1, g1, be1, w2, b2, g2, be2):
    def body(xi_ref, xs_ref, w1a_ref, w1b_ref, b1_ref, g1_ref, be1_ref,
             w2_ref, b2_ref, g2_ref, be2_ref, out_ref):
        h = jnp.dot(xi_ref[...], w1a_ref[...], preferred_element_type=jnp.float32)
        h = h + jnp.dot(xs_ref[...], w1b_ref[...], preferred_element_type=jnp.float32)
        h = h + b1_ref[...]
        mu = jnp.mean(h, axis=-1, keepdims=True)
        c = h - mu
        var = jnp.mean(c * c, axis=-1, keepdims=True)
        h = c * lax.rsqrt(var + EPS) * g1_ref[...] + be1_ref[...]
        h = 0.5 * h * (1.0 + lax.erf(h * _INV_SQRT2))
        h = jnp.dot(h, w2_ref[...], preferred_element_type=jnp.float32) + b2_ref[...]
        mu = jnp.mean(h, axis=-1, keepdims=True)
        c = h - mu
        var = jnp.mean(c * c, axis=-1, keepdims=True)
        h = c * lax.rsqrt(var + EPS) * g2_ref[...] + be2_ref[...]
        out_ref[...] = 0.5 * h * (1.0 + lax.erf(h * _INV_SQRT2))

    row_spec = pl.BlockSpec((RB, D), lambda i: (i, 0))
    full = lambda shape: pl.BlockSpec(shape, lambda i: (0,) * len(shape))
    return pl.pallas_call(
        body,
        grid=(N // RB,),
        in_specs=[
            row_spec,
            row_spec,
            full((D, D)), full((D, D)), full((1, D)), full((1, D)), full((1, D)),
            full((D, D)), full((1, D)), full((1, D)), full((1, D)),
        ],
        out_specs=row_spec,
        out_shape=jax.ShapeDtypeStruct((N, D), jnp.float32),
    )(xi, xs, w1a, w1b, b1, g1, be1, w2, b2, g2, be2)


def kernel(x_down, x_skip, upsample_idx, W1, b1, g1, be1, W2, b2, g2, be2):
    idx = upsample_idx.astype(jnp.int32)
    idx = jnp.concatenate([idx, jnp.zeros((NP - N,), jnp.int32)])
    xi = _sc_gather(x_down, idx.reshape(NW, NCH, CH))
    return _tc_mlp(
        xi, x_skip,
        W1[:D], W1[D:],
        b1.reshape(1, D), g1.reshape(1, D), be1.reshape(1, D),
        W2, b2.reshape(1, D), g2.reshape(1, D), be2.reshape(1, D),
    )
